# baseline (speedup 1.0000x reference)
import functools

import jax
import jax.numpy as jnp
from jax import lax
from jax.experimental import pallas as pl
from jax.experimental.pallas import tpu as pltpu
from jax.experimental.pallas import tpu_sc as plsc

N = 10000
E = 320000
D = 128
DH = 64
DO = 16
D_OUT = 7
NP = 10240
W = 128
W32 = 80
W16 = 160
NBUF = 8
NB1 = 5
NB2 = 8
E_PAD = 32 * W32 * W
RPT = NP // 16

_mesh = plsc.VectorSubcoreMesh(core_axis_name="c", subcore_axis_name="s")
_no_tc_tiling = pltpu.CompilerParams(use_tc_tiling_on_sc=False)


@functools.partial(
    pl.kernel,
    out_type=jax.ShapeDtypeStruct((2, NP), jnp.float32),
    mesh=_mesh,
    compiler_params=_no_tc_tiling,
    scratch_types=[
        pltpu.VMEM((W32 * W,), jnp.int32),
        [pltpu.VMEM((W,), jnp.int32)] * NBUF,
        pltpu.VMEM((W,), jnp.float32),
        pltpu.VMEM_SHARED((NP,), jnp.float32),
        pltpu.SemaphoreType.DMA,
        [pltpu.SemaphoreType.DMA] * NBUF,
    ],
)
def _sc_degree(dst_hbm, zeros_hbm, out_hbm, didx_all, didxb, ones_v, acc,
               sem_i, sems_s):
    c = lax.axis_index("c")
    s = lax.axis_index("s")
    gw = c * 16 + s
    my_rows = pl.ds(s * RPT, RPT)

    for j in range(W // 16):
        ones_v[pl.ds(j * 16, 16)] = jnp.ones((16,), jnp.float32)

    cp_i = pltpu.async_copy(dst_hbm.at[pl.ds(gw * W32 * W, W32 * W)],
                            didx_all, sem_i)
    pltpu.sync_copy(zeros_hbm.at[my_rows], acc.at[my_rows])
    cp_i.wait()
    plsc.subcore_barrier()

    def copy_idx_row(v, dst_ref):
        for j in range(W // 16):
            dst_ref[pl.ds(j * 16, 16)] = didx_all[pl.ds(v * W + j * 16, 16)]

    for b in range(NBUF):
        copy_idx_row(b, didxb[b])

    @pl.loop(0, W32 // NBUF)
    def _(g):
        cps = [pltpu.async_copy(ones_v, acc.at[didxb[b]], sems_s[b], add=True)
               for b in range(NBUF)]
        for b in range(NBUF):
            cps[b].wait()
            nxt = g * NBUF + b + NBUF

            @pl.when(nxt < W32)
            def _():
                copy_idx_row(nxt, didxb[b])

    plsc.subcore_barrier()
    pltpu.sync_copy(acc.at[my_rows], out_hbm.at[c].at[my_rows])


@functools.partial(
    pl.kernel,
    out_type=jax.ShapeDtypeStruct((NP, D), jnp.float32),
    mesh=_mesh,
    compiler_params=_no_tc_tiling,
    scratch_types=[
        pltpu.VMEM((W16 * W,), jnp.int32),
        pltpu.VMEM((W16 * W,), jnp.int32),
        [pltpu.VMEM((W,), jnp.int32)] * NB1,
        [pltpu.VMEM((W,), jnp.int32)] * NB1,
        [pltpu.VMEM((W, DH), jnp.float32)] * NB1,
        pltpu.VMEM_SHARED((NP, DH), jnp.float32),
        [pltpu.SemaphoreType.DMA] * NB1,
        [pltpu.SemaphoreType.DMA] * NB1,
        pltpu.SemaphoreType.DMA,
    ],
)
def _sc_agg1(g_hbm, src_hbm, dst_hbm, zeros_hbm, out_hbm,
             sidx_all, didx_all, sidxb, didxb, rows, acc,
             sems_g, sems_s, sem_i):
    c = lax.axis_index("c")
    s = lax.axis_index("s")
    my_rows = pl.ds(s * RPT, RPT)

    cp_s = pltpu.async_copy(src_hbm.at[pl.ds(s * W16 * W, W16 * W)],
                            sidx_all, sem_i)
    cp_d = pltpu.async_copy(dst_hbm.at[pl.ds(s * W16 * W, W16 * W)],
                            didx_all, sem_i)
    pltpu.sync_copy(zeros_hbm.at[my_rows], acc.at[my_rows])
    cp_s.wait()
    cp_d.wait()
    plsc.subcore_barrier()

    def stage_idx(v, b):
        for j in range(W // 16):
            sidxb[b][pl.ds(j * 16, 16)] = (
                sidx_all[pl.ds(v * W + j * 16, 16)] * 2 + c)
            didxb[b][pl.ds(j * 16, 16)] = didx_all[pl.ds(v * W + j * 16, 16)]

    for b in range(NB1):
        stage_idx(b, b)
        pltpu.async_copy(g_hbm.at[sidxb[b]], rows[b], sems_g[b])

    @pl.loop(0, W16 // NB1)
    def _(g):
        cps = []
        for b in range(NB1):
            pltpu.make_async_copy(g_hbm.at[sidxb[b]], rows[b],
                                  sems_g[b]).wait()
            cps.append(pltpu.async_copy(rows[b], acc.at[didxb[b]],
                                        sems_s[b], add=True))
        for b in range(NB1):
            cps[b].wait()
            nxt = g * NB1 + b + NB1

            @pl.when(nxt < W16)
            def _():
                stage_idx(nxt, b)
                pltpu.async_copy(g_hbm.at[sidxb[b]], rows[b], sems_g[b])

    plsc.subcore_barrier()
    pltpu.sync_copy(acc.at[my_rows], out_hbm.at[my_rows, pl.ds(c * DH, DH)])


DH2 = DO // 2


@functools.partial(
    pl.kernel,
    out_type=jax.ShapeDtypeStruct((NP, DO), jnp.float32),
    mesh=_mesh,
    compiler_params=_no_tc_tiling,
    scratch_types=[
        pltpu.VMEM((W16 * W,), jnp.int32),
        pltpu.VMEM((W16 * W,), jnp.int32),
        [pltpu.VMEM((W,), jnp.int32)] * NB2,
        [pltpu.VMEM((W,), jnp.int32)] * NB2,
        [pltpu.VMEM((W, DH2), jnp.float32)] * NB2,
        pltpu.VMEM_SHARED((NP, DH2), jnp.float32),
        [pltpu.SemaphoreType.DMA] * NB2,
        [pltpu.SemaphoreType.DMA] * NB2,
        pltpu.SemaphoreType.DMA,
    ],
)
def _sc_agg2(g_hbm, src_hbm, dst_hbm, zeros_hbm, out_hbm,
             sidx_all, didx_all, sidxb, didxb, rows, acc,
             sems_g, sems_s, sem_i):
    c = lax.axis_index("c")
    s = lax.axis_index("s")
    my_rows = pl.ds(s * RPT, RPT)

    cp_s = pltpu.async_copy(src_hbm.at[pl.ds(s * W16 * W, W16 * W)],
                            sidx_all, sem_i)
    cp_d = pltpu.async_copy(dst_hbm.at[pl.ds(s * W16 * W, W16 * W)],
                            didx_all, sem_i)
    pltpu.sync_copy(zeros_hbm.at[my_rows], acc.at[my_rows])
    cp_s.wait()
    cp_d.wait()
    plsc.subcore_barrier()

    def stage_idx(v, b):
        for j in range(W // 16):
            sidxb[b][pl.ds(j * 16, 16)] = (
                sidx_all[pl.ds(v * W + j * 16, 16)] * 2 + c)
            didxb[b][pl.ds(j * 16, 16)] = didx_all[pl.ds(v * W + j * 16, 16)]

    for b in range(NB2):
        stage_idx(b, b)
        pltpu.async_copy(g_hbm.at[sidxb[b]], rows[b], sems_g[b])

    @pl.loop(0, W16 // NB2)
    def _(g):
        cps = []
        for b in range(NB2):
            pltpu.make_async_copy(g_hbm.at[sidxb[b]], rows[b],
                                  sems_g[b]).wait()
            cps.append(pltpu.async_copy(rows[b], acc.at[didxb[b]],
                                        sems_s[b], add=True))
        for b in range(NB2):
            cps[b].wait()
            nxt = g * NB2 + b + NB2

            @pl.when(nxt < W16)
            def _():
                stage_idx(nxt, b)
                pltpu.async_copy(g_hbm.at[sidxb[b]], rows[b], sems_g[b])

    plsc.subcore_barrier()
    pltpu.sync_copy(acc.at[my_rows], out_hbm.at[my_rows, pl.ds(c * DH2, DH2)])


_BLK = 1024
_GRID = NP // _BLK
_DR = _BLK // 128


def _tc_b_body(deg_ref, x_ref, w1_ref, g1_ref, dinv_ref):
    deg = deg_ref[0] + deg_ref[1] + 1.0
    dinv = lax.rsqrt(deg)
    h1 = jnp.dot(x_ref[...], w1_ref[...], preferred_element_type=jnp.float32)
    g1_ref[...] = dinv * h1
    dinv_ref[...] = dinv


def _tc_d_body(dinv_ref, agg_ref, g1_ref, b1_ref, w2_ref, g2_ref):
    i = pl.program_id(0)
    dinv = dinv_ref[...]
    out1 = dinv * (agg_ref[...] + g1_ref[...]) + b1_ref[...][None, :]
    r = jnp.maximum(out1, 0.0)
    h2 = jnp.dot(r, w2_ref[...], preferred_element_type=jnp.float32)
    g2 = dinv * h2
    row = i * _BLK + lax.broadcasted_iota(jnp.int32, (_BLK, DO), 0)
    g2_ref[...] = jnp.where(row < N, g2, 0.0)


def _tc_f_body(dinv_ref, agg_ref, g2_ref, b2_ref, out_ref):
    z = dinv_ref[...] * (agg_ref[...] + g2_ref[...]) + b2_ref[...][None, :]
    lane = lax.broadcasted_iota(jnp.int32, (_BLK, DO), 1)
    z = jnp.where(lane < D_OUT, z, -1e30)
    m = jnp.max(z, axis=1, keepdims=True)
    lse = jnp.log(jnp.sum(jnp.exp(z - m), axis=1, keepdims=True)) + m
    out_ref[...] = (z - lse)[:, :D_OUT]


def kernel(x, edge_index, W1, b1, W2, b2):
    src = edge_index[0]
    dst = edge_index[1]
    npad = E_PAD - E
    pad_idx = (N + (jnp.arange(npad, dtype=jnp.int32) % (NP - N))).astype(jnp.int32)
    srcp = jnp.concatenate([src, pad_idx])
    dstp = jnp.concatenate([dst, pad_idx])

    xp = jnp.pad(x, ((0, NP - N), (0, 0)))
    w2p = jnp.pad(W2, ((0, 0), (0, DO - D_OUT)))
    b2p = jnp.pad(b2, (0, DO - D_OUT))
    z1 = jnp.zeros((NP,), jnp.float32)
    z64 = jnp.zeros((NP, DH), jnp.float32)
    z8 = jnp.zeros((NP, DH2), jnp.float32)

    degp = _sc_degree(dstp, z1)
    degp3 = degp.reshape(2, NP, 1)

    g1, dinv = pl.pallas_call(
        _tc_b_body,
        grid=(_GRID,),
        in_specs=[
            pl.BlockSpec((2, _BLK, 1), lambda i: (0, i, 0)),
            pl.BlockSpec((_BLK, D), lambda i: (i, 0)),
            pl.BlockSpec((D, D), lambda i: (0, 0)),
        ],
        out_specs=[
            pl.BlockSpec((_BLK, D), lambda i: (i, 0)),
            pl.BlockSpec((_BLK, 1), lambda i: (i, 0)),
        ],
        out_shape=[
            jax.ShapeDtypeStruct((NP, D), jnp.float32),
            jax.ShapeDtypeStruct((NP, 1), jnp.float32),
        ],
    )(degp3, xp, W1)

    gstack = g1.reshape(2 * NP, DH)
    agg1 = _sc_agg1(gstack, srcp, dstp, z64)

    g2 = pl.pallas_call(
        _tc_d_body,
        grid=(_GRID,),
        in_specs=[
            pl.BlockSpec((_BLK, 1), lambda i: (i, 0)),
            pl.BlockSpec((_BLK, D), lambda i: (i, 0)),
            pl.BlockSpec((_BLK, D), lambda i: (i, 0)),
            pl.BlockSpec((D,), lambda i: (0,)),
            pl.BlockSpec((D, DO), lambda i: (0, 0)),
        ],
        out_specs=pl.BlockSpec((_BLK, DO), lambda i: (i, 0)),
        out_shape=jax.ShapeDtypeStruct((NP, DO), jnp.float32),
    )(dinv, agg1, g1, b1, w2p)

    gstack2 = g2.reshape(2 * NP, DH2)
    agg2 = _sc_agg2(gstack2, srcp, dstp, z8)

    out = pl.pallas_call(
        _tc_f_body,
        grid=(_GRID,),
        in_specs=[
            pl.BlockSpec((_BLK, 1), lambda i: (i, 0)),
            pl.BlockSpec((_BLK, DO), lambda i: (i, 0)),
            pl.BlockSpec((_BLK, DO), lambda i: (i, 0)),
            pl.BlockSpec((DO,), lambda i: (0,)),
        ],
        out_specs=pl.BlockSpec((_BLK, D_OUT), lambda i: (i, 0)),
        out_shape=jax.ShapeDtypeStruct((N, D_OUT), jnp.float32),
    )(dinv, agg2, g2, b2p)

    return out


# speedup vs baseline: 1.0673x; 1.0673x over previous
import functools

import jax
import jax.numpy as jnp
from jax import lax
from jax.experimental import pallas as pl
from jax.experimental.pallas import tpu as pltpu
from jax.experimental.pallas import tpu_sc as plsc

N = 10000
E = 320000
D = 128
DH = 64
DO = 16
D_OUT = 7
NP = 10240
W = 128
W32 = 80
W16 = 160
NBUF = 8
NB1 = 5
NB2 = 8
E_PAD = 32 * W32 * W
RPT = NP // 16

_mesh = plsc.VectorSubcoreMesh(core_axis_name="c", subcore_axis_name="s")
_no_tc_tiling = pltpu.CompilerParams(use_tc_tiling_on_sc=False)


@functools.partial(
    pl.kernel,
    out_type=jax.ShapeDtypeStruct((2, NP), jnp.float32),
    mesh=_mesh,
    compiler_params=_no_tc_tiling,
    scratch_types=[
        pltpu.VMEM((W32 * W,), jnp.int32),
        [pltpu.VMEM((W,), jnp.int32)] * NBUF,
        pltpu.VMEM((W,), jnp.float32),
        pltpu.VMEM_SHARED((NP,), jnp.float32),
        pltpu.SemaphoreType.DMA,
        [pltpu.SemaphoreType.DMA] * NBUF,
    ],
)
def _sc_degree(dst_hbm, zeros_hbm, out_hbm, didx_all, didxb, ones_v, acc,
               sem_i, sems_s):
    c = lax.axis_index("c")
    s = lax.axis_index("s")
    gw = c * 16 + s
    my_rows = pl.ds(s * RPT, RPT)

    for j in range(W // 16):
        ones_v[pl.ds(j * 16, 16)] = jnp.ones((16,), jnp.float32)

    cp_i = pltpu.async_copy(dst_hbm.at[pl.ds(gw * W32 * W, W32 * W)],
                            didx_all, sem_i)
    pltpu.sync_copy(zeros_hbm.at[my_rows], acc.at[my_rows])
    cp_i.wait()
    plsc.subcore_barrier()

    def copy_idx_row(v, dst_ref):
        for j in range(W // 16):
            dst_ref[pl.ds(j * 16, 16)] = didx_all[pl.ds(v * W + j * 16, 16)]

    for b in range(NBUF):
        copy_idx_row(b, didxb[b])

    @pl.loop(0, W32 // NBUF)
    def _(g):
        cps = [pltpu.async_copy(ones_v, acc.at[didxb[b]], sems_s[b], add=True)
               for b in range(NBUF)]
        for b in range(NBUF):
            cps[b].wait()
            nxt = g * NBUF + b + NBUF

            @pl.when(nxt < W32)
            def _():
                copy_idx_row(nxt, didxb[b])

    plsc.subcore_barrier()
    pltpu.sync_copy(acc.at[my_rows], out_hbm.at[c].at[my_rows])


@functools.partial(
    pl.kernel,
    out_type=jax.ShapeDtypeStruct((NP, D), jnp.float32),
    mesh=_mesh,
    compiler_params=_no_tc_tiling,
    scratch_types=[
        pltpu.VMEM((W16 * W,), jnp.int32),
        pltpu.VMEM((W16 * W,), jnp.int32),
        [pltpu.VMEM((W,), jnp.int32)] * NB1,
        [pltpu.VMEM((W,), jnp.int32)] * NB1,
        [pltpu.VMEM((W, DH), jnp.float32)] * NB1,
        pltpu.VMEM_SHARED((NP, DH), jnp.float32),
        [pltpu.SemaphoreType.DMA] * NB1,
        [pltpu.SemaphoreType.DMA] * NB1,
        pltpu.SemaphoreType.DMA,
    ],
)
def _sc_agg1(g_hbm, src_hbm, dst_hbm, zeros_hbm, out_hbm,
             sidx_all, didx_all, sidxb, didxb, rows, acc,
             sems_g, sems_s, sem_i):
    c = lax.axis_index("c")
    s = lax.axis_index("s")
    my_rows = pl.ds(s * RPT, RPT)

    cp_s = pltpu.async_copy(src_hbm.at[pl.ds(s * W16 * W, W16 * W)],
                            sidx_all, sem_i)
    cp_d = pltpu.async_copy(dst_hbm.at[pl.ds(s * W16 * W, W16 * W)],
                            didx_all, sem_i)
    pltpu.sync_copy(zeros_hbm.at[my_rows], acc.at[my_rows])
    cp_s.wait()
    cp_d.wait()
    plsc.subcore_barrier()

    def stage_idx(v, b):
        for j in range(W // 16):
            sidxb[b][pl.ds(j * 16, 16)] = (
                sidx_all[pl.ds(v * W + j * 16, 16)] * 2 + c)
            didxb[b][pl.ds(j * 16, 16)] = didx_all[pl.ds(v * W + j * 16, 16)]

    for b in range(NB1):
        stage_idx(b, b)
        pltpu.async_copy(g_hbm.at[sidxb[b]], rows[b], sems_g[b])

    @pl.loop(0, W16 // NB1)
    def _(g):
        cps = []
        for b in range(NB1):
            pltpu.make_async_copy(g_hbm.at[sidxb[b]], rows[b],
                                  sems_g[b]).wait()
            cps.append(pltpu.async_copy(rows[b], acc.at[didxb[b]],
                                        sems_s[b], add=True))
        for b in range(NB1):
            cps[b].wait()
            nxt = g * NB1 + b + NB1

            @pl.when(nxt < W16)
            def _():
                stage_idx(nxt, b)
                pltpu.async_copy(g_hbm.at[sidxb[b]], rows[b], sems_g[b])

    plsc.subcore_barrier()
    pltpu.sync_copy(acc.at[my_rows], out_hbm.at[my_rows, pl.ds(c * DH, DH)])


@functools.partial(
    pl.kernel,
    out_type=jax.ShapeDtypeStruct((2, NP, DO), jnp.float32),
    mesh=_mesh,
    compiler_params=_no_tc_tiling,
    scratch_types=[
        pltpu.VMEM((W32 * W,), jnp.int32),
        pltpu.VMEM((W32 * W,), jnp.int32),
        [pltpu.VMEM((W,), jnp.int32)] * NB2,
        [pltpu.VMEM((W,), jnp.int32)] * NB2,
        [pltpu.VMEM((W, DO), jnp.float32)] * NB2,
        pltpu.VMEM_SHARED((NP, DO), jnp.float32),
        [pltpu.SemaphoreType.DMA] * NB2,
        [pltpu.SemaphoreType.DMA] * NB2,
        pltpu.SemaphoreType.DMA,
    ],
)
def _sc_agg2(g_hbm, src_hbm, dst_hbm, zeros_hbm, out_hbm,
             sidx_all, didx_all, sidxb, didxb, rows, acc,
             sems_g, sems_s, sem_i):
    c = lax.axis_index("c")
    s = lax.axis_index("s")
    gw = c * 16 + s
    my_rows = pl.ds(s * RPT, RPT)

    cp_s = pltpu.async_copy(src_hbm.at[pl.ds(gw * W32 * W, W32 * W)],
                            sidx_all, sem_i)
    cp_d = pltpu.async_copy(dst_hbm.at[pl.ds(gw * W32 * W, W32 * W)],
                            didx_all, sem_i)
    pltpu.sync_copy(zeros_hbm.at[my_rows], acc.at[my_rows])
    cp_s.wait()
    cp_d.wait()
    plsc.subcore_barrier()

    def stage_idx(v, b):
        for j in range(W // 16):
            sidxb[b][pl.ds(j * 16, 16)] = sidx_all[pl.ds(v * W + j * 16, 16)]
            didxb[b][pl.ds(j * 16, 16)] = didx_all[pl.ds(v * W + j * 16, 16)]

    for b in range(NB2):
        stage_idx(b, b)
        pltpu.async_copy(g_hbm.at[sidxb[b]], rows[b], sems_g[b])

    @pl.loop(0, W32 // NB2)
    def _(g):
        cps = []
        for b in range(NB2):
            pltpu.make_async_copy(g_hbm.at[sidxb[b]], rows[b],
                                  sems_g[b]).wait()
            cps.append(pltpu.async_copy(rows[b], acc.at[didxb[b]],
                                        sems_s[b], add=True))
        for b in range(NB2):
            cps[b].wait()
            nxt = g * NB2 + b + NB2

            @pl.when(nxt < W32)
            def _():
                stage_idx(nxt, b)
                pltpu.async_copy(g_hbm.at[sidxb[b]], rows[b], sems_g[b])

    plsc.subcore_barrier()
    pltpu.sync_copy(acc.at[my_rows], out_hbm.at[c].at[my_rows])


_BLK = 1024
_GRID = NP // _BLK
_DR = _BLK // 128


def _tc_b_body(deg_ref, x_ref, w1_ref, g1_ref, dinv_ref):
    deg = deg_ref[0] + deg_ref[1] + 1.0
    dinv = lax.rsqrt(deg)
    h1 = jnp.dot(x_ref[...], w1_ref[...], preferred_element_type=jnp.float32)
    g1_ref[...] = dinv * h1
    dinv_ref[...] = dinv


def _tc_d_body(dinv_ref, agg_ref, g1_ref, b1_ref, w2_ref, g2_ref):
    i = pl.program_id(0)
    dinv = dinv_ref[...]
    out1 = dinv * (agg_ref[...] + g1_ref[...]) + b1_ref[...][None, :]
    r = jnp.maximum(out1, 0.0)
    h2 = jnp.dot(r, w2_ref[...], preferred_element_type=jnp.float32)
    g2 = dinv * h2
    row = i * _BLK + lax.broadcasted_iota(jnp.int32, (_BLK, DO), 0)
    g2_ref[...] = jnp.where(row < N, g2, 0.0)


def _tc_f_body(dinv_ref, agg_ref, g2_ref, b2_ref, out_ref):
    a = agg_ref[0] + agg_ref[1]
    z = dinv_ref[...] * (a + g2_ref[...]) + b2_ref[...][None, :]
    lane = lax.broadcasted_iota(jnp.int32, (_BLK, DO), 1)
    z = jnp.where(lane < D_OUT, z, -1e30)
    m = jnp.max(z, axis=1, keepdims=True)
    lse = jnp.log(jnp.sum(jnp.exp(z - m), axis=1, keepdims=True)) + m
    out_ref[...] = (z - lse)[:, :D_OUT]


def kernel(x, edge_index, W1, b1, W2, b2):
    src = edge_index[0]
    dst = edge_index[1]
    npad = E_PAD - E
    pad_idx = (N + (jnp.arange(npad, dtype=jnp.int32) % (NP - N))).astype(jnp.int32)
    srcp = jnp.concatenate([src, pad_idx])
    dstp = jnp.concatenate([dst, pad_idx])

    xp = jnp.pad(x, ((0, NP - N), (0, 0)))
    w2p = jnp.pad(W2, ((0, 0), (0, DO - D_OUT)))
    b2p = jnp.pad(b2, (0, DO - D_OUT))
    z1 = jnp.zeros((NP,), jnp.float32)
    z64 = jnp.zeros((NP, DH), jnp.float32)
    z16 = jnp.zeros((NP, DO), jnp.float32)

    degp = _sc_degree(dstp, z1)
    degp3 = degp.reshape(2, NP, 1)

    g1, dinv = pl.pallas_call(
        _tc_b_body,
        grid=(_GRID,),
        in_specs=[
            pl.BlockSpec((2, _BLK, 1), lambda i: (0, i, 0)),
            pl.BlockSpec((_BLK, D), lambda i: (i, 0)),
            pl.BlockSpec((D, D), lambda i: (0, 0)),
        ],
        out_specs=[
            pl.BlockSpec((_BLK, D), lambda i: (i, 0)),
            pl.BlockSpec((_BLK, 1), lambda i: (i, 0)),
        ],
        out_shape=[
            jax.ShapeDtypeStruct((NP, D), jnp.float32),
            jax.ShapeDtypeStruct((NP, 1), jnp.float32),
        ],
    )(degp3, xp, W1)

    gstack = g1.reshape(2 * NP, DH)
    agg1 = _sc_agg1(gstack, srcp, dstp, z64)

    g2 = pl.pallas_call(
        _tc_d_body,
        grid=(_GRID,),
        in_specs=[
            pl.BlockSpec((_BLK, 1), lambda i: (i, 0)),
            pl.BlockSpec((_BLK, D), lambda i: (i, 0)),
            pl.BlockSpec((_BLK, D), lambda i: (i, 0)),
            pl.BlockSpec((D,), lambda i: (0,)),
            pl.BlockSpec((D, DO), lambda i: (0, 0)),
        ],
        out_specs=pl.BlockSpec((_BLK, DO), lambda i: (i, 0)),
        out_shape=jax.ShapeDtypeStruct((NP, DO), jnp.float32),
    )(dinv, agg1, g1, b1, w2p)

    agg2 = _sc_agg2(g2, srcp, dstp, z16)

    out = pl.pallas_call(
        _tc_f_body,
        grid=(_GRID,),
        in_specs=[
            pl.BlockSpec((_BLK, 1), lambda i: (i, 0)),
            pl.BlockSpec((2, _BLK, DO), lambda i: (0, i, 0)),
            pl.BlockSpec((_BLK, DO), lambda i: (i, 0)),
            pl.BlockSpec((DO,), lambda i: (0,)),
        ],
        out_specs=pl.BlockSpec((_BLK, D_OUT), lambda i: (i, 0)),
        out_shape=jax.ShapeDtypeStruct((N, D_OUT), jnp.float32),
    )(dinv, agg2, g2, b2p)

    return out
